# baseline (speedup 1.0000x reference)
import functools

import jax
import jax.numpy as jnp
from jax import lax
from jax.experimental import pallas as pl
from jax.experimental.pallas import tpu as pltpu
from jax.experimental.pallas import tpu_sc as plsc

N_NODES = 10000
N_EDGES = 320000
HID = 32
BN_EPS = 1e-5

NC = 2
NS = 16
NW = NC * NS
E_PER_W = N_EDGES // NW
EBLK = 128
NBLK_W = 78
ETAIL = E_PER_W - NBLK_W * EBLK
ROWS_SUB = 640
N_PAD = ROWS_SUB * NS
NBUF = 6

def _sc_body(z_hbm, ei_hbm, out_hbm,
             src_v, dst_v, rows_v, stage_v, agg_sh, z_sh, gsem, ssem):
    c = lax.axis_index("c")
    s = lax.axis_index("s")
    wid = c * NS + s

    zrows = N_NODES // NS
    pltpu.sync_copy(z_hbm.at[pl.ds(s * zrows, zrows)],
                    z_sh.at[pl.ds(s * zrows, zrows)])

    zeros16 = jnp.zeros((16,), jnp.float32)

    def _zero(i, carry):
        stage_v[i, pl.ds(0, 16)] = zeros16
        stage_v[i, pl.ds(16, 16)] = zeros16
        return carry

    lax.fori_loop(0, ROWS_SUB, _zero, 0)
    pltpu.sync_copy(stage_v, agg_sh.at[pl.ds(s * ROWS_SUB, ROWS_SUB)])

    ebase = wid * E_PER_W
    pltpu.sync_copy(ei_hbm.at[0, pl.ds(ebase, E_PER_W)], src_v)
    pltpu.sync_copy(ei_hbm.at[1, pl.ds(ebase, E_PER_W)], dst_v)
    plsc.subcore_barrier()

    for b in range(NBUF):
        pltpu.async_copy(z_sh.at[src_v.at[pl.ds(b * EBLK, EBLK)]],
                         rows_v.at[b], gsem.at[b])

    def _group(g, carry):
        j0 = g * NBUF
        descs = []
        for b in range(NBUF):
            j = j0 + b
            pltpu.make_async_copy(z_sh.at[src_v.at[pl.ds(j * EBLK, EBLK)]],
                                  rows_v.at[b], gsem.at[b]).wait()
            descs.append(pltpu.async_copy(
                rows_v.at[b],
                agg_sh.at[dst_v.at[pl.ds(j * EBLK, EBLK)]],
                ssem.at[b], add=True))
        for b in range(NBUF):
            j = j0 + b
            descs[b].wait()

            @pl.when(j + NBUF < NBLK_W)
            def _():
                pltpu.async_copy(
                    z_sh.at[src_v.at[pl.ds((j + NBUF) * EBLK, EBLK)]],
                    rows_v.at[b], gsem.at[b])
        return carry

    lax.fori_loop(0, NBLK_W // NBUF, _group, 0)

    tbase = NBLK_W * EBLK
    pltpu.async_copy(z_sh.at[src_v.at[pl.ds(tbase, ETAIL)]],
                     rows_v.at[0, pl.ds(0, ETAIL)], gsem.at[0]).wait()
    pltpu.sync_copy(rows_v.at[0, pl.ds(0, ETAIL)],
                    agg_sh.at[dst_v.at[pl.ds(tbase, ETAIL)]], add=True)
    plsc.subcore_barrier()

    pltpu.sync_copy(agg_sh.at[pl.ds(s * ROWS_SUB, ROWS_SUB)],
                    out_hbm.at[c, pl.ds(s * ROWS_SUB, ROWS_SUB)])


@functools.cache
def _get_sc_kernel():
    mesh = plsc.VectorSubcoreMesh(core_axis_name="c", subcore_axis_name="s")
    return pl.kernel(
        _sc_body,
        out_type=jax.ShapeDtypeStruct((NC, N_PAD, HID), jnp.float32),
        mesh=mesh,
        scratch_types=[
            pltpu.VMEM((E_PER_W,), jnp.int32),
            pltpu.VMEM((E_PER_W,), jnp.int32),
            pltpu.VMEM((NBUF, EBLK, HID), jnp.float32),
            pltpu.VMEM((ROWS_SUB, HID), jnp.float32),
            pltpu.VMEM_SHARED((N_PAD, HID), jnp.float32),
            pltpu.VMEM_SHARED((N_NODES, HID), jnp.float32),
            pltpu.SemaphoreType.DMA((NBUF,)),
            pltpu.SemaphoreType.DMA((NBUF,)),
        ],
        compiler_params=pltpu.CompilerParams(use_tc_tiling_on_sc=False),
    )


def _sc_segment_sum(z, edge_index):
    return _get_sc_kernel()(z, edge_index)


N_PK = N_NODES // 4
NP_PK = N_PAD // 4


def _tile4(v):
    return jnp.concatenate([v, v, v, v])


def _fold4_mean(v):
    return (v[0:32] + v[32:64] + v[64:96] + v[96:128]) * 0.25


def _kron4(w):
    zb = jnp.zeros((32, 32), jnp.float32)
    rows = []
    for i in range(4):
        blocks = [zb] * i + [w] + [zb] * (3 - i)
        rows.append(jnp.concatenate(blocks, axis=1))
    return jnp.concatenate(rows, axis=0)


def _bn_packed(x, gamma, beta):
    mu = _tile4(_fold4_mean(jnp.mean(x, axis=0)))
    xc = x - mu
    var = _tile4(_fold4_mean(jnp.mean(xc * xc, axis=0)))
    return _tile4(gamma) * xc * lax.rsqrt(var + BN_EPS) + _tile4(beta)


def _tc_head_body(h_ref, w1_ref, wp_ref, bp_ref, z_ref, s_ref):
    hmat = h_ref[...]
    z_ref[...] = jnp.dot(hmat, w1_ref[...], preferred_element_type=jnp.float32)
    pooled = jnp.sum(hmat, axis=0, keepdims=True)
    s_ref[...] = (jnp.dot(pooled, wp_ref[...],
                          preferred_element_type=jnp.float32) + bp_ref[...])


def _tc_layer_body(*refs, has_next):
    (z_ref, agg_ref, b0,
     w1, b1, w2, b2, w3, b3, w4, b4,
     g0, e0, g1, e1, g2, e2, g3, e3,
     ga, ea, go, eo) = refs[:23]
    rest = refs[23:]
    if has_next:
        wn, wp, bp, zn_ref, s_ref = rest
    else:
        wp, bp, s_ref = rest

    y = z_ref[...] + agg_ref[0, :N_PK] + agg_ref[1, :N_PK] + _tile4(b0[...])
    y = jax.nn.relu(_bn_packed(y, g0[...], e0[...]))
    for w, bv, g, e in ((w1, b1, g1, e1), (w2, b2, g2, e2), (w3, b3, g3, e3)):
        y = (jnp.dot(y, _kron4(w[...]), preferred_element_type=jnp.float32)
             + _tile4(bv[...]))
        y = jax.nn.relu(_bn_packed(y, g[...], e[...]))
    y = (jnp.dot(y, _kron4(w4[...]), preferred_element_type=jnp.float32)
         + _tile4(b4[...]))
    x = jax.nn.relu(_bn_packed(y, ga[...], ea[...]))
    x = jax.nn.relu(_bn_packed(x, go[...], eo[...]))
    if has_next:
        zn_ref[...] = jnp.dot(x, _kron4(wn[...]),
                              preferred_element_type=jnp.float32)
    colsum = jnp.sum(x, axis=0)
    pooled = (colsum[0:32] + colsum[32:64] + colsum[64:96] + colsum[96:128])
    s_ref[...] = (jnp.dot(pooled[None, :], wp[...],
                          preferred_element_type=jnp.float32) + bp[...])


def _apply_layer(z, aggp, lp, w1_next, wp, bp):
    lins = lp["mlp"]["lins"]
    bns = lp["mlp"]["bns"]
    args = [z, aggp, lins[0][1]]
    for i in range(1, 5):
        args += [lins[i][0], lins[i][1]]
    for i in range(4):
        args += [bns[i][0], bns[i][1]]
    args += [lp["bn_apply"][0], lp["bn_apply"][1],
             lp["bn_outer"][0], lp["bn_outer"][1]]
    has_next = w1_next is not None
    if has_next:
        args.append(w1_next)
    args += [wp, bp]
    out_shape = [jax.ShapeDtypeStruct((1, 16), jnp.float32)]
    if has_next:
        out_shape = [jax.ShapeDtypeStruct((N_PK, 128), jnp.float32)] + out_shape
    return pl.pallas_call(
        functools.partial(_tc_layer_body, has_next=has_next),
        out_shape=out_shape,
    )(*args)


def kernel(h, edge_index, params):
    gin = params["gin"]
    pred = params["pred"]

    z_flat, score = pl.pallas_call(
        _tc_head_body,
        out_shape=[jax.ShapeDtypeStruct((N_NODES, HID), jnp.float32),
                   jax.ShapeDtypeStruct((1, 16), jnp.float32)],
    )(h, gin[0]["mlp"]["lins"][0][0], pred[0][0], pred[0][1])
    z_pk = z_flat.reshape(N_PK, 128)

    for l in range(3):
        aggp = _sc_segment_sum(z_flat, edge_index)
        agg_pk = aggp.reshape(NC, NP_PK, 128)
        w1_next = gin[l + 1]["mlp"]["lins"][0][0] if l < 2 else None
        outs = _apply_layer(z_pk, agg_pk, gin[l], w1_next,
                            pred[l + 1][0], pred[l + 1][1])
        if l < 2:
            z_pk, s = outs
            z_flat = z_pk.reshape(N_NODES, HID)
        else:
            (s,) = outs
        score = score + s
    return score


# speedup vs baseline: 1.2513x; 1.2513x over previous
import functools

import jax
import jax.numpy as jnp
from jax import lax
from jax.experimental import pallas as pl
from jax.experimental.pallas import tpu as pltpu
from jax.experimental.pallas import tpu_sc as plsc

N_NODES = 10000
N_EDGES = 320000
HID = 32
BN_EPS = 1e-5

NC = 2
NS = 16
NW = NC * NS
E_PER_W = N_EDGES // NW
EBLK = 128
NBLK_W = 78
ETAIL = E_PER_W - NBLK_W * EBLK
ROWS_SUB = 640
N_PAD = ROWS_SUB * NS
NBUF = 13

def _sc_body(z_hbm, ei_hbm, out_hbm,
             src_v, dst_v, rows_v, stage_v, agg_sh, gsem, ssem):
    c = lax.axis_index("c")
    s = lax.axis_index("s")
    wid = c * NS + s

    zeros16 = jnp.zeros((16,), jnp.float32)

    def _zero(i, carry):
        stage_v[i, pl.ds(0, 16)] = zeros16
        stage_v[i, pl.ds(16, 16)] = zeros16
        return carry

    lax.fori_loop(0, ROWS_SUB, _zero, 0)
    pltpu.sync_copy(stage_v, agg_sh.at[pl.ds(s * ROWS_SUB, ROWS_SUB)])

    ebase = wid * E_PER_W
    pltpu.sync_copy(ei_hbm.at[0, pl.ds(ebase, E_PER_W)], src_v)
    pltpu.sync_copy(ei_hbm.at[1, pl.ds(ebase, E_PER_W)], dst_v)
    plsc.subcore_barrier()

    for b in range(NBUF):
        pltpu.async_copy(z_hbm.at[src_v.at[pl.ds(b * EBLK, EBLK)]],
                         rows_v.at[b], gsem.at[b])

    def _group(g, carry):
        j0 = g * NBUF
        descs = []
        for b in range(NBUF):
            j = j0 + b
            pltpu.make_async_copy(z_hbm.at[src_v.at[pl.ds(j * EBLK, EBLK)]],
                                  rows_v.at[b], gsem.at[b]).wait()
            descs.append(pltpu.async_copy(
                rows_v.at[b],
                agg_sh.at[dst_v.at[pl.ds(j * EBLK, EBLK)]],
                ssem.at[b], add=True))
        for b in range(NBUF):
            j = j0 + b
            descs[b].wait()

            @pl.when(j + NBUF < NBLK_W)
            def _():
                pltpu.async_copy(
                    z_hbm.at[src_v.at[pl.ds((j + NBUF) * EBLK, EBLK)]],
                    rows_v.at[b], gsem.at[b])
        return carry

    lax.fori_loop(0, NBLK_W // NBUF, _group, 0)

    tbase = NBLK_W * EBLK
    pltpu.async_copy(z_hbm.at[src_v.at[pl.ds(tbase, ETAIL)]],
                     rows_v.at[0, pl.ds(0, ETAIL)], gsem.at[0]).wait()
    pltpu.sync_copy(rows_v.at[0, pl.ds(0, ETAIL)],
                    agg_sh.at[dst_v.at[pl.ds(tbase, ETAIL)]], add=True)
    plsc.subcore_barrier()

    pltpu.sync_copy(agg_sh.at[pl.ds(s * ROWS_SUB, ROWS_SUB)],
                    out_hbm.at[c, pl.ds(s * ROWS_SUB, ROWS_SUB)])


@functools.cache
def _get_sc_kernel():
    mesh = plsc.VectorSubcoreMesh(core_axis_name="c", subcore_axis_name="s")
    return pl.kernel(
        _sc_body,
        out_type=jax.ShapeDtypeStruct((NC, N_PAD, HID), jnp.float32),
        mesh=mesh,
        scratch_types=[
            pltpu.VMEM((E_PER_W,), jnp.int32),
            pltpu.VMEM((E_PER_W,), jnp.int32),
            pltpu.VMEM((NBUF, EBLK, HID), jnp.float32),
            pltpu.VMEM((ROWS_SUB, HID), jnp.float32),
            pltpu.VMEM_SHARED((N_PAD, HID), jnp.float32),
            pltpu.SemaphoreType.DMA((NBUF,)),
            pltpu.SemaphoreType.DMA((NBUF,)),
        ],
        compiler_params=pltpu.CompilerParams(use_tc_tiling_on_sc=False),
    )


def _sc_segment_sum(z, edge_index):
    return _get_sc_kernel()(z, edge_index)


N_PK = N_NODES // 4
NP_PK = N_PAD // 4


def _tile4(v):
    return jnp.concatenate([v, v, v, v])


def _fold4_mean(v):
    return (v[0:32] + v[32:64] + v[64:96] + v[96:128]) * 0.25


def _kron4(w):
    zb = jnp.zeros((32, 32), jnp.float32)
    rows = []
    for i in range(4):
        blocks = [zb] * i + [w] + [zb] * (3 - i)
        rows.append(jnp.concatenate(blocks, axis=1))
    return jnp.concatenate(rows, axis=0)


def _bn_packed(x, gamma, beta):
    mu = _tile4(_fold4_mean(jnp.mean(x, axis=0)))
    xc = x - mu
    var = _tile4(_fold4_mean(jnp.mean(xc * xc, axis=0)))
    return _tile4(gamma) * xc * lax.rsqrt(var + BN_EPS) + _tile4(beta)


def _tc_head_body(h_ref, w1_ref, wp_ref, bp_ref, z_ref, s_ref):
    hmat = h_ref[...]
    z_ref[...] = jnp.dot(hmat, w1_ref[...], preferred_element_type=jnp.float32)
    pooled = jnp.sum(hmat, axis=0, keepdims=True)
    s_ref[...] = (jnp.dot(pooled, wp_ref[...],
                          preferred_element_type=jnp.float32) + bp_ref[...])


def _tc_layer_body(*refs, has_next):
    (z_ref, agg_ref, b0,
     w1, b1, w2, b2, w3, b3, w4, b4,
     g0, e0, g1, e1, g2, e2, g3, e3,
     ga, ea, go, eo) = refs[:23]
    rest = refs[23:]
    if has_next:
        wn, wp, bp, zn_ref, s_ref = rest
    else:
        wp, bp, s_ref = rest

    y = z_ref[...] + agg_ref[0, :N_PK] + agg_ref[1, :N_PK] + _tile4(b0[...])
    y = jax.nn.relu(_bn_packed(y, g0[...], e0[...]))
    for w, bv, g, e in ((w1, b1, g1, e1), (w2, b2, g2, e2), (w3, b3, g3, e3)):
        y = (jnp.dot(y, _kron4(w[...]), preferred_element_type=jnp.float32)
             + _tile4(bv[...]))
        y = jax.nn.relu(_bn_packed(y, g[...], e[...]))
    y = (jnp.dot(y, _kron4(w4[...]), preferred_element_type=jnp.float32)
         + _tile4(b4[...]))
    x = jax.nn.relu(_bn_packed(y, ga[...], ea[...]))
    x = jax.nn.relu(_bn_packed(x, go[...], eo[...]))
    if has_next:
        zn_ref[...] = jnp.dot(x, _kron4(wn[...]),
                              preferred_element_type=jnp.float32)
    colsum = jnp.sum(x, axis=0)
    pooled = (colsum[0:32] + colsum[32:64] + colsum[64:96] + colsum[96:128])
    s_ref[...] = (jnp.dot(pooled[None, :], wp[...],
                          preferred_element_type=jnp.float32) + bp[...])


def _apply_layer(z, aggp, lp, w1_next, wp, bp):
    lins = lp["mlp"]["lins"]
    bns = lp["mlp"]["bns"]
    args = [z, aggp, lins[0][1]]
    for i in range(1, 5):
        args += [lins[i][0], lins[i][1]]
    for i in range(4):
        args += [bns[i][0], bns[i][1]]
    args += [lp["bn_apply"][0], lp["bn_apply"][1],
             lp["bn_outer"][0], lp["bn_outer"][1]]
    has_next = w1_next is not None
    if has_next:
        args.append(w1_next)
    args += [wp, bp]
    out_shape = [jax.ShapeDtypeStruct((1, 16), jnp.float32)]
    if has_next:
        out_shape = [jax.ShapeDtypeStruct((N_PK, 128), jnp.float32)] + out_shape
    return pl.pallas_call(
        functools.partial(_tc_layer_body, has_next=has_next),
        out_shape=out_shape,
    )(*args)


def kernel(h, edge_index, params):
    gin = params["gin"]
    pred = params["pred"]

    z_flat, score = pl.pallas_call(
        _tc_head_body,
        out_shape=[jax.ShapeDtypeStruct((N_NODES, HID), jnp.float32),
                   jax.ShapeDtypeStruct((1, 16), jnp.float32)],
    )(h, gin[0]["mlp"]["lins"][0][0], pred[0][0], pred[0][1])
    z_pk = z_flat.reshape(N_PK, 128)

    for l in range(3):
        aggp = _sc_segment_sum(z_flat, edge_index)
        agg_pk = aggp.reshape(NC, NP_PK, 128)
        w1_next = gin[l + 1]["mlp"]["lins"][0][0] if l < 2 else None
        outs = _apply_layer(z_pk, agg_pk, gin[l], w1_next,
                            pred[l + 1][0], pred[l + 1][1])
        if l < 2:
            z_pk, s = outs
            z_flat = z_pk.reshape(N_NODES, HID)
        else:
            (s,) = outs
        score = score + s
    return score
